# baseline (speedup 1.0000x reference)
import functools

import jax
import jax.numpy as jnp
from jax import lax
from jax.experimental import pallas as pl
from jax.experimental.pallas import tpu as pltpu
from jax.experimental.pallas import tpu_sc as plsc

_TOT = 4800000
_NW = 30
_CPW = _TOT // _NW
_PIECE = 32000
_NPIECE = _CPW // _PIECE
_NGRP = 48


@functools.partial(
    pl.kernel,
    out_type=jax.ShapeDtypeStruct((_TOT,), jnp.float32),
    mesh=plsc.VectorSubcoreMesh(core_axis_name="c", subcore_axis_name="s"),
    scratch_types=[
        pltpu.VMEM((16,), jnp.float32),
        pltpu.VMEM((_NGRP * 16,), jnp.float32),
        pltpu.VMEM((2, _PIECE), jnp.float32),
        pltpu.SemaphoreType.DMA((2,)),
        pltpu.SemaphoreType.DMA((2,)),
    ],
)
def _sc_op(x_hbm, c_hbm, o_hbm, vbuf, cbuf, bufs, sin, sout):
    wid = lax.axis_index("s") * 2 + lax.axis_index("c")
    base = wid * _CPW

    @pl.when(wid < _NW)
    def _():
        ins = [
            pltpu.make_async_copy(
                x_hbm.at[pl.ds(base + p * _PIECE, _PIECE)], bufs.at[p % 2], sin.at[p % 2]
            )
            for p in range(_NPIECE)
        ]
        outs = [
            pltpu.make_async_copy(
                bufs.at[p % 2], o_hbm.at[pl.ds(base + p * _PIECE, _PIECE)], sout.at[p % 2]
            )
            for p in range(_NPIECE)
        ]
        ins[0].start()
        for p in range(_NPIECE):
            if p + 1 < _NPIECE:
                if p >= 1:
                    outs[p - 1].wait()
                ins[p + 1].start()
            ins[p].wait()
            outs[p].start()
        outs[_NPIECE - 2].wait()
        outs[_NPIECE - 1].wait()
        pltpu.sync_copy(c_hbm, cbuf)
        for g in range(_NGRP):
            off = g * 100000

            @pl.when(wid == off // _CPW)
            def _():
                pltpu.sync_copy(x_hbm.at[pl.ds(off, 16)], vbuf)
                vbuf[...] = vbuf[...] + cbuf[pl.ds(g * 16, 16)]
                pltpu.sync_copy(vbuf, o_hbm.at[pl.ds(off, 16)])


def kernel(x):
    t = jnp.arange(0, 336, 1, dtype=jnp.float32).reshape(7, 6, 8)
    addvals = t[jnp.array([0, 2, 1, 3, 4, 5, 6])]
    ctab = jnp.zeros((6, 8, 16), jnp.float32).at[:, :, 0:7].set(
        addvals.transpose(1, 2, 0)
    ).reshape(_NGRP * 16)
    xt = jnp.transpose(x, (1, 2, 0)).reshape(_TOT)
    res = _sc_op(xt, ctab)
    return jnp.transpose(res.reshape(6, 8, 100000), (2, 0, 1))


# speedup vs baseline: 7.4257x; 7.4257x over previous
import jax
import jax.numpy as jnp
import numpy as np
from jax.experimental import pallas as pl
from jax.experimental.pallas import tpu as pltpu

_N = 100000
_BL = 50048
_GRID = -(-_N // _BL)

_T = np.arange(0, 336, 1, dtype=np.float32).reshape(7, 6, 8)
_CADD_T = np.zeros((6, 8, 128), np.float32)
_CADD_T[:, :, 0:7] = _T[[0, 2, 1, 3, 4, 5, 6]].transpose(1, 2, 0)


def _body(x_ref, c_ref, o_ref):
    o_ref[...] = x_ref[...]
    @pl.when(pl.program_id(0) == 0)
    def _():
        o_ref[:, :, 0:128] = o_ref[:, :, 0:128] + c_ref[...]


def kernel(x):
    caddT = jnp.asarray(_CADD_T)
    xt = jnp.transpose(x, (1, 2, 0))
    res = pl.pallas_call(
        _body,
        grid=(_GRID,),
        in_specs=[
            pl.BlockSpec((6, 8, _BL), lambda i: (0, 0, i)),
            pl.BlockSpec((6, 8, 128), lambda i: (0, 0, 0)),
        ],
        out_specs=pl.BlockSpec((6, 8, _BL), lambda i: (0, 0, i)),
        out_shape=jax.ShapeDtypeStruct((6, 8, _N), jnp.float32),
        compiler_params=pltpu.CompilerParams(
            dimension_semantics=("arbitrary",),
        ),
    )(xt, caddT)
    return jnp.transpose(res, (2, 0, 1))
